# baseline (speedup 1.0000x reference)
import functools

import jax
import jax.numpy as jnp
from jax import lax
from jax.experimental import pallas as pl
from jax.experimental.pallas import tpu as pltpu
from jax.experimental.pallas import tpu_sc as plsc

_NC = 2
_NS = 16
_NW = _NC * _NS
_LANES = 16

_CHUNK = 128
_IDXROW = 128
_NSUB = _CHUNK // _IDXROW


@functools.lru_cache(maxsize=None)
def _build(M, N, D):
    assert M % (_NW * _CHUNK) == 0
    per_w = M // _NW
    n_chunks = per_w // _CHUNK
    t_piece = per_w // 8
    scale = jnp.float32(N - 1)
    vregs_per_row = _IDXROW // _LANES

    mesh = plsc.VectorSubcoreMesh(
        core_axis_name="c", subcore_axis_name="s",
        num_cores=_NC, num_subcores=_NS)

    @functools.partial(
        pl.kernel,
        out_type=jax.ShapeDtypeStruct((M, D), jnp.float32),
        mesh=mesh,
        compiler_params=pltpu.CompilerParams(use_tc_tiling_on_sc=False),
        scratch_types=[
            pltpu.VMEM((t_piece // _IDXROW, _IDXROW), jnp.float32),
            pltpu.VMEM((per_w // _IDXROW, _IDXROW), jnp.int32),
            pltpu.VMEM((per_w // _IDXROW, _IDXROW), jnp.int32),
            pltpu.VMEM((per_w + _LANES,), jnp.float32),
            [pltpu.VMEM((_CHUNK, D), jnp.float32) for _ in range(2)],
            [pltpu.VMEM((_CHUNK, D), jnp.float32) for _ in range(2)],
            [pltpu.VMEM((_CHUNK, D), jnp.float32) for _ in range(2)],
            [pltpu.SemaphoreType.DMA for _ in range(2)],
            [pltpu.SemaphoreType.DMA for _ in range(2)],
        ],
    )
    def table_lerp(t_hbm, values_hbm, out_hbm,
                   t_v, i0_v, i1_v, a_v, r0_v, r1_v, o_v, gsem, osem):
        wid = lax.axis_index("s") * _NC + lax.axis_index("c")
        base_w = wid * per_w

        def piece_body(h, _):
            row0 = pl.multiple_of((base_w + h * t_piece) // _IDXROW,
                                  t_piece // _IDXROW)
            pltpu.sync_copy(t_hbm.at[pl.ds(row0, t_piece // _IDXROW)], t_v)

            def index_body(vi, _):
                vg = h * (t_piece // _LANES) + vi
                row = vg // vregs_per_row
                col = (vg % vregs_per_row) * _LANES
                tt = t_v[vi // vregs_per_row,
                         pl.ds((vi % vregs_per_row) * _LANES, _LANES)]
                x = tt * scale
                k = jnp.clip(x.astype(jnp.int32), 0, N - 2)
                a = jnp.clip(x - k.astype(jnp.float32), 0.0, 1.0)
                i0_v[row, pl.ds(col, _LANES)] = k * 2
                i1_v[row, pl.ds(col, _LANES)] = k * 2 + 1
                a_v[pl.ds(vg * _LANES, _LANES)] = a
                return 0

            lax.fori_loop(0, t_piece // _LANES, index_body, 0)
            return 0

        lax.fori_loop(0, 8, piece_body, 0)

        def fire_gathers(ci, b):
            for j in range(_NSUB):
                row = ci * _NSUB + j
                dst = pl.ds(j * _IDXROW, _IDXROW)
                pltpu.async_copy(values_hbm.at[i0_v.at[row]],
                                 r0_v[b].at[dst], gsem[b])
                pltpu.async_copy(values_hbm.at[i1_v.at[row]],
                                 r1_v[b].at[dst], gsem[b])

        def wait_gathers(b):
            for j in range(_NSUB):
                dst = pl.ds(j * _IDXROW, _IDXROW)
                pltpu.make_async_copy(values_hbm.at[i0_v.at[0]],
                                      r0_v[b].at[dst], gsem[b]).wait()
                pltpu.make_async_copy(values_hbm.at[i0_v.at[0]],
                                      r1_v[b].at[dst], gsem[b]).wait()

        def lerp(ci, rb, ob):
            group = 8

            def lerp_body(qb, _):
                q0 = qb * group
                av16 = a_v[pl.ds(ci * _CHUNK + q0, _LANES)]
                for lane in range(group):
                    q = q0 + lane
                    av = jnp.broadcast_to(av16[lane], (_LANES,))
                    for part in range(D // _LANES):
                        sl = pl.ds(part * _LANES, _LANES)
                        r0 = r0_v[rb][q, sl]
                        r1 = r1_v[rb][q, sl]
                        o_v[ob][q, sl] = r0 + av * (r1 - r0)
                return 0

            lax.fori_loop(0, _CHUNK // group, lerp_body, 0)

        def out_slice(ci):
            base = pl.multiple_of(base_w + ci * _CHUNK, _CHUNK)
            return out_hbm.at[pl.ds(base, _CHUNK)]

        fire_gathers(0, 0)
        fire_gathers(1, 1)

        def pair_body(cp, _):
            for b in range(2):
                ci = cp * 2 + b
                wait_gathers(b)

                @pl.when(cp >= 1)
                def _():
                    pltpu.make_async_copy(o_v[b], out_slice(ci - 2),
                                          osem[b]).wait()

                lerp(ci, b, b)
                pltpu.async_copy(o_v[b], out_slice(ci), osem[b])

                @pl.when(cp < (n_chunks // 2) - 1)
                def _():
                    fire_gathers(ci + 2, b)
            return 0

        lax.fori_loop(0, n_chunks // 2, pair_body, 0)

        for b in range(2):
            pltpu.make_async_copy(o_v[b], out_slice(n_chunks - 2 + b),
                                  osem[b]).wait()

    return table_lerp


def kernel(t, values, t_grid):
    M = t.shape[0]
    N, D = values.shape
    vnext = jnp.concatenate([values[1:], values[-1:]], axis=0)
    vv = jnp.concatenate([values, vnext], axis=1)
    t2 = t.reshape(M // 128, 128)
    return _build(M, N, D)(t2, vv.reshape(2 * N, D))


# speedup vs baseline: 1.1635x; 1.1635x over previous
import functools

import jax
import jax.numpy as jnp
from jax import lax
from jax.experimental import pallas as pl
from jax.experimental.pallas import tpu as pltpu
from jax.experimental.pallas import tpu_sc as plsc

_NC = 2
_NS = 16
_NW = _NC * _NS
_LANES = 16

_CHUNK = 128
_IDXROW = 128
_NSUB = _CHUNK // _IDXROW


@functools.lru_cache(maxsize=None)
def _build(M, N, D):
    assert M % (_NW * _CHUNK) == 0
    per_w = M // _NW
    n_chunks = per_w // _CHUNK
    t_piece = per_w // 8
    scale = jnp.float32(N - 1)
    vregs_per_row = _IDXROW // _LANES

    mesh = plsc.VectorSubcoreMesh(
        core_axis_name="c", subcore_axis_name="s",
        num_cores=_NC, num_subcores=_NS)

    @functools.partial(
        pl.kernel,
        out_type=jax.ShapeDtypeStruct((M, D), jnp.float32),
        mesh=mesh,
        compiler_params=pltpu.CompilerParams(use_tc_tiling_on_sc=True),
        scratch_types=[
            pltpu.VMEM((t_piece // _IDXROW, _IDXROW), jnp.float32),
            pltpu.VMEM((per_w // _IDXROW, _IDXROW), jnp.int32),
            pltpu.VMEM((per_w + _LANES,), jnp.float32),
            [pltpu.VMEM((_CHUNK, 2 * D), jnp.float32) for _ in range(4)],
            [pltpu.VMEM((_CHUNK, D), jnp.float32) for _ in range(2)],
            [pltpu.SemaphoreType.DMA for _ in range(4)],
            [pltpu.SemaphoreType.DMA for _ in range(2)],
        ],
    )
    def table_lerp(t_hbm, values_hbm, out_hbm,
                   t_v, i0_v, a_v, rv_v, o_v, gsem, osem):
        wid = lax.axis_index("s") * _NC + lax.axis_index("c")
        base_w = wid * per_w

        def piece_body(h, _):
            row0 = pl.multiple_of((base_w + h * t_piece) // _IDXROW,
                                  t_piece // _IDXROW)
            pltpu.sync_copy(t_hbm.at[pl.ds(row0, t_piece // _IDXROW)], t_v)

            def index_body(vi, _):
                vg = h * (t_piece // _LANES) + vi
                row = vg // vregs_per_row
                col = (vg % vregs_per_row) * _LANES
                tt = t_v[vi // vregs_per_row,
                         pl.ds((vi % vregs_per_row) * _LANES, _LANES)]
                x = tt * scale
                k = jnp.clip(x.astype(jnp.int32), 0, N - 2)
                a = jnp.clip(x - k.astype(jnp.float32), 0.0, 1.0)
                i0_v[row, pl.ds(col, _LANES)] = k
                a_v[pl.ds(vg * _LANES, _LANES)] = a
                return 0

            lax.fori_loop(0, t_piece // _LANES, index_body, 0)
            return 0

        lax.fori_loop(0, 8, piece_body, 0)

        def fire_gathers(ci, b):
            for j in range(_NSUB):
                row = ci * _NSUB + j
                dst = pl.ds(j * _IDXROW, _IDXROW)
                pltpu.async_copy(values_hbm.at[i0_v.at[row]],
                                 rv_v[b].at[dst], gsem[b])

        def wait_gathers(b):
            for j in range(_NSUB):
                dst = pl.ds(j * _IDXROW, _IDXROW)
                pltpu.make_async_copy(values_hbm.at[i0_v.at[0]],
                                      rv_v[b].at[dst], gsem[b]).wait()

        def lerp(ci, rb, ob):
            group = 8

            def lerp_body(qb, _):
                q0 = qb * group
                av16 = a_v[pl.ds(ci * _CHUNK + q0, _LANES)]
                for lane in range(group):
                    q = q0 + lane
                    av = jnp.broadcast_to(av16[lane], (_LANES,))
                    for part in range(D // _LANES):
                        r0 = rv_v[rb][q, pl.ds(part * _LANES, _LANES)]
                        r1 = rv_v[rb][q, pl.ds(D + part * _LANES, _LANES)]
                        o_v[ob][q, pl.ds(part * _LANES, _LANES)] = (
                            r0 + av * (r1 - r0))
                return 0

            lax.fori_loop(0, _CHUNK // group, lerp_body, 0)

        def out_slice(ci):
            base = pl.multiple_of(base_w + ci * _CHUNK, _CHUNK)
            return out_hbm.at[pl.ds(base, _CHUNK)]

        for b in range(4):
            fire_gathers(b, b)

        def quad_body(cq, _):
            for b in range(4):
                ci = cq * 4 + b
                ob = b % 2
                wait_gathers(b)

                if b >= 2:
                    pltpu.make_async_copy(o_v[ob], out_slice(ci - 2),
                                          osem[ob]).wait()
                else:
                    @pl.when(cq >= 1)
                    def _():
                        pltpu.make_async_copy(o_v[ob], out_slice(ci - 2),
                                              osem[ob]).wait()

                lerp(ci, b, ob)
                pltpu.async_copy(o_v[ob], out_slice(ci), osem[ob])

                @pl.when(cq < (n_chunks // 4) - 1)
                def _():
                    fire_gathers(ci + 4, b)
            return 0

        lax.fori_loop(0, n_chunks // 4, quad_body, 0)

        for b in range(2):
            pltpu.make_async_copy(o_v[b], out_slice(n_chunks - 2 + b),
                                  osem[b]).wait()

    return table_lerp


def kernel(t, values, t_grid):
    M = t.shape[0]
    N, D = values.shape
    vnext = jnp.concatenate([values[1:], values[-1:]], axis=0)
    vv = jnp.concatenate([values, vnext], axis=1)
    t2 = t.reshape(M // 128, 128)
    return _build(M, N, D)(t2, vv)
